# Initial kernel scaffold; baseline (speedup 1.0000x reference)
import jax
import jax.numpy as jnp
from jax.experimental import pallas as pl


def kernel(x, edge_index, edge_type, local_global_id, ent_emb, rel_emb):
    raise NotImplementedError("write your pallas kernel here")


# speedup vs baseline: 7.2766x; 7.2766x over previous
import functools

import jax
import jax.numpy as jnp
from jax import lax
from jax.experimental import pallas as pl
from jax.experimental.pallas import tpu as pltpu
from jax.experimental.pallas import tpu_sc as plsc

_N_LOCAL = 10000
_E = 320000
_D = 128
_R = 237

_NC = 2
_NS = 16
_NW = _NC * _NS
_EPW = _E // _NW
_C = 80
_NCHUNK = _EPW // _C
_GRP = _C // 16

_mesh = plsc.VectorSubcoreMesh(core_axis_name="c", subcore_axis_name="s")


@functools.partial(
    pl.kernel,
    mesh=_mesh,
    out_type=jax.ShapeDtypeStruct((_E,), jnp.float32),
    compiler_params=pltpu.CompilerParams(needs_layout_passes=False),
    scratch_types=[
        pltpu.VMEM((_N_LOCAL,), jnp.int32),
        pltpu.VMEM((_C,), jnp.int32),
        pltpu.VMEM((_C,), jnp.int32),
        pltpu.VMEM((_C,), jnp.int32),
        pltpu.VMEM((_C,), jnp.int32),
        pltpu.VMEM((_C, _D), jnp.float32),
        pltpu.VMEM((_C, _D), jnp.float32),
        pltpu.VMEM((_C, _D), jnp.float32),
        pltpu.VMEM((17 * 16,), jnp.float32),
        pltpu.VMEM((_C,), jnp.float32),
        pltpu.SemaphoreType.DMA,
        pltpu.SemaphoreType.DMA,
        pltpu.SemaphoreType.DMA,
    ],
)
def _sc_score(head_hbm, tail_hbm, rtype_hbm, lgid_hbm, x_hbm, ent_hbm,
              rel_hbm, out_hbm,
              lgid_v, hidx_v, tidx_v, ridx_v, gidx_v,
              hrow_v, trow_v, rrow_v, psum_v, out_v,
              sem0, sem1, sem2):
    wid = lax.axis_index("s") * _NC + lax.axis_index("c")
    base = wid * _EPW
    pltpu.sync_copy(lgid_hbm, lgid_v)
    iota = lax.iota(jnp.int32, 16)

    def chunk_body(c, carry):
        off = base + c * _C
        pltpu.sync_copy(head_hbm.at[pl.ds(off, _C)], hidx_v)
        pltpu.sync_copy(tail_hbm.at[pl.ds(off, _C)], tidx_v)
        pltpu.sync_copy(rtype_hbm.at[pl.ds(off, _C)], ridx_v)
        for j in range(_GRP):
            t = tidx_v[pl.ds(16 * j, 16)]
            gidx_v[pl.ds(16 * j, 16)] = plsc.load_gather(lgid_v, [t])
        cp0 = pltpu.async_copy(x_hbm.at[hidx_v], hrow_v, sem0)
        cp1 = pltpu.async_copy(ent_hbm.at[gidx_v], trow_v, sem1)
        cp2 = pltpu.async_copy(rel_hbm.at[ridx_v], rrow_v, sem2)
        cp0.wait()
        cp1.wait()
        cp2.wait()

        def grp_body(gi, inner_carry):
            for l in range(16):
                e = gi * 16 + l
                acc = None
                for k in range(8):
                    h = hrow_v[e, pl.ds(16 * k, 16)]
                    t = trow_v[e, pl.ds(16 * k, 16)]
                    r = rrow_v[e, pl.ds(16 * k, 16)]
                    v = jnp.abs(h + r - t)
                    acc = v if acc is None else acc + v
                plsc.store_scatter(psum_v, [iota * 17 + l], acc)
            sc = None
            for i in range(16):
                vi = plsc.load_gather(psum_v, [iota + 17 * i])
                sc = vi if sc is None else sc + vi
            out_v[pl.ds(gi * 16, 16)] = sc
            return inner_carry

        lax.fori_loop(0, _GRP, grp_body, 0)
        pltpu.sync_copy(out_v, out_hbm.at[pl.ds(off, _C)])
        return carry

    lax.fori_loop(0, _NCHUNK, chunk_body, 0)


def kernel(x, edge_index, edge_type, local_global_id, ent_emb, rel_emb):
    head = edge_index[0]
    tail = edge_index[1]
    return _sc_score(head, tail, edge_type, local_global_id, x, ent_emb,
                     rel_emb)


# speedup vs baseline: 15.0698x; 2.0710x over previous
import functools

import jax
import jax.numpy as jnp
from jax import lax
from jax.experimental import pallas as pl
from jax.experimental.pallas import tpu as pltpu
from jax.experimental.pallas import tpu_sc as plsc

_N_LOCAL = 10000
_E = 320000
_D = 128
_R = 237

_NC = 2
_NS = 16
_NW = _NC * _NS
_EPW = _E // _NW
_C = 80
_NCHUNK = _EPW // _C
_GRP = _C // 16

_mesh = plsc.VectorSubcoreMesh(core_axis_name="c", subcore_axis_name="s")


@functools.partial(
    pl.kernel,
    mesh=_mesh,
    out_type=jax.ShapeDtypeStruct((_E,), jnp.float32),
    compiler_params=pltpu.CompilerParams(needs_layout_passes=False),
    scratch_types=[
        pltpu.VMEM((_N_LOCAL,), jnp.int32),
        pltpu.VMEM((2, _C), jnp.int32),
        pltpu.VMEM((2, _C), jnp.int32),
        pltpu.VMEM((2, _C), jnp.int32),
        pltpu.VMEM((2, _C), jnp.int32),
        pltpu.VMEM((2, _C, _D), jnp.float32),
        pltpu.VMEM((2, _C, _D), jnp.float32),
        pltpu.VMEM((2, _C, _D), jnp.float32),
        pltpu.VMEM((17 * 16,), jnp.float32),
        pltpu.VMEM((2, _C), jnp.float32),
        pltpu.SemaphoreType.DMA((2,)),
        pltpu.SemaphoreType.DMA((2,)),
        pltpu.SemaphoreType.DMA((2,)),
        pltpu.SemaphoreType.DMA((2,)),
        pltpu.SemaphoreType.DMA((2,)),
        pltpu.SemaphoreType.DMA((2,)),
        pltpu.SemaphoreType.DMA((2,)),
    ],
)
def _sc_score(head_hbm, tail_hbm, rtype_hbm, lgid_hbm, x_hbm, ent_hbm,
              rel_hbm, out_hbm,
              lgid_v, hidx_v, tidx_v, ridx_v, gidx_v,
              hrow_v, trow_v, rrow_v, psum_v, out_v,
              semih, semit, semir, semgh, semgt, semgr, semo):
    wid = lax.axis_index("s") * _NC + lax.axis_index("c")
    base = wid * _EPW
    pltpu.sync_copy(lgid_hbm, lgid_v)
    iota = lax.iota(jnp.int32, 16)

    def issue_idx(c, p, guard=False):
        off = base + c * _C

        def go():
            pltpu.async_copy(head_hbm.at[pl.ds(off, _C)], hidx_v.at[p],
                             semih.at[p])
            pltpu.async_copy(tail_hbm.at[pl.ds(off, _C)], tidx_v.at[p],
                             semit.at[p])
            pltpu.async_copy(rtype_hbm.at[pl.ds(off, _C)], ridx_v.at[p],
                             semir.at[p])

        if guard:
            pl.when(c < _NCHUNK)(go)
        else:
            go()

    def stage(p):
        pltpu.make_async_copy(head_hbm.at[pl.ds(0, _C)], hidx_v.at[p],
                              semih.at[p]).wait()
        pltpu.make_async_copy(tail_hbm.at[pl.ds(0, _C)], tidx_v.at[p],
                              semit.at[p]).wait()
        pltpu.make_async_copy(rtype_hbm.at[pl.ds(0, _C)], ridx_v.at[p],
                              semir.at[p]).wait()
        for j in range(_GRP):
            t = tidx_v[p, pl.ds(16 * j, 16)]
            gidx_v[p, pl.ds(16 * j, 16)] = plsc.load_gather(lgid_v, [t])
        pltpu.async_copy(x_hbm.at[hidx_v.at[p]], hrow_v.at[p], semgh.at[p])
        pltpu.async_copy(ent_hbm.at[gidx_v.at[p]], trow_v.at[p], semgt.at[p])
        pltpu.async_copy(rel_hbm.at[ridx_v.at[p]], rrow_v.at[p], semgr.at[p])

    def wait_gathers(p):
        pltpu.make_async_copy(x_hbm.at[hidx_v.at[p]], hrow_v.at[p],
                              semgh.at[p]).wait()
        pltpu.make_async_copy(ent_hbm.at[gidx_v.at[p]], trow_v.at[p],
                              semgt.at[p]).wait()
        pltpu.make_async_copy(rel_hbm.at[ridx_v.at[p]], rrow_v.at[p],
                              semgr.at[p]).wait()

    def wait_out(p):
        pltpu.make_async_copy(out_v.at[p], out_hbm.at[pl.ds(0, _C)],
                              semo.at[p]).wait()

    def compute(c, p):
        def grp_body(gi, carry):
            for l in range(16):
                e = gi * 16 + l
                acc = None
                for k in range(8):
                    h = hrow_v[p, e, pl.ds(16 * k, 16)]
                    t = trow_v[p, e, pl.ds(16 * k, 16)]
                    r = rrow_v[p, e, pl.ds(16 * k, 16)]
                    v = jnp.abs(h + r - t)
                    acc = v if acc is None else acc + v
                plsc.store_scatter(psum_v, [iota * 17 + l], acc)
            sc = None
            for i in range(16):
                vi = plsc.load_gather(psum_v, [iota + 17 * i])
                sc = vi if sc is None else sc + vi
            out_v[p, pl.ds(gi * 16, 16)] = sc
            return carry

        lax.fori_loop(0, _GRP, grp_body, 0)
        off = base + c * _C
        pltpu.async_copy(out_v.at[p], out_hbm.at[pl.ds(off, _C)], semo.at[p])

    def run_iter(c, p, first=False, stage_next=True):
        wait_gathers(p)
        if stage_next:
            issue_idx(c + 2, p, guard=True)
            stage(1 - p)
        if not first:
            wait_out(p)
        compute(c, p)

    issue_idx(0, 0)
    stage(0)
    issue_idx(1, 1)
    run_iter(0, 0, first=True)
    run_iter(1, 1, first=True)

    def pair_body(i, carry):
        c = 2 + 2 * i
        run_iter(c, 0)
        run_iter(c + 1, 1)
        return carry

    lax.fori_loop(0, (_NCHUNK - 3) // 2, pair_body, 0)

    run_iter(_NCHUNK - 1, 0, stage_next=False)
    wait_out(0)
    wait_out(1)


def kernel(x, edge_index, edge_type, local_global_id, ent_emb, rel_emb):
    head = edge_index[0]
    tail = edge_index[1]
    return _sc_score(head, tail, edge_type, local_global_id, x, ent_emb,
                     rel_emb)
